# Initial kernel scaffold; baseline (speedup 1.0000x reference)
import jax
import jax.numpy as jnp
from jax.experimental import pallas as pl


def kernel(x, edge_index, node_depth, batch_index, W1, b1, W2, b2, conv_w, conv_b, fc1_w, fc1_b, fc2_w, fc2_b):
    raise NotImplementedError("write your pallas kernel here")


# speedup vs baseline: 19.4724x; 19.4724x over previous
import jax
import jax.numpy as jnp
from jax import lax
from jax.experimental import pallas as pl
from jax.experimental.pallas import tpu as pltpu
from jax.experimental.pallas import tpu_sc as plsc

N = 100000
E = 1600000
IN_DIM = 2
H = 64
CC = 64
LVL = 128
NB = 4
SEG = NB * LVL
SEGP = 528

NC = 2
NS = 16
NW = NC * NS

Q = 16
NQ = 4

WSUB = 125
RSUB = 8
CHUNK = RSUB * WSUB

ROWS_PER_TILE = N // NS
ZROWS = 625

NP = 102400
NODES_PER_TILE = NP // NW
PCHUNK = 640

_mesh = plsc.VectorSubcoreMesh(core_axis_name="c", subcore_axis_name="s")


def _zero_fill(buf, nrows):
  def zb(i, carry):
    buf[i, :] = jnp.zeros((Q,), jnp.float32)
    return carry
  lax.fori_loop(0, nrows, zb, None)


def _zero_acc(acc, zbuf, s):
  for p in range(ROWS_PER_TILE // ZROWS):
    pltpu.sync_copy(zbuf, acc.at[pl.ds(s * ROWS_PER_TILE + p * ZROWS, ZROWS)])


def _dump_acc(acc, outp, q, s):
  for p in range(ROWS_PER_TILE // ZROWS):
    off = s * ROWS_PER_TILE + p * ZROWS
    pltpu.sync_copy(acc.at[pl.ds(off, ZROWS)], outp.at[q, pl.ds(off, ZROWS)])


def _sc_deg(dst_rs):
  ept = E // NC // NS
  nch = ept // CHUNK

  def body(dst_rs, outp, acc, didx, ones_rows, zbuf):
    c = lax.axis_index("c")
    s = lax.axis_index("s")
    _zero_fill(zbuf, ZROWS)
    def ob(i, carry):
      ones_rows[i, :] = (1 - jnp.minimum(lax.iota(jnp.int32, Q), 1)).astype(jnp.float32)
      return carry
    lax.fori_loop(0, WSUB, ob, None)
    _zero_acc(acc, zbuf, s)
    plsc.subcore_barrier()
    row0 = (c * (E // NC) + s * ept) // WSUB
    def chunk(ch, carry):
      r0 = row0 + ch * (CHUNK // WSUB)
      pltpu.sync_copy(dst_rs.at[pl.ds(r0, RSUB)], didx)
      for r in range(RSUB):
        pltpu.sync_copy(ones_rows, acc.at[didx.at[r]], add=True)
      return carry
    lax.fori_loop(0, nch, chunk, None)
    plsc.subcore_barrier()
    _dump_acc(acc, outp, c, s)

  return pl.kernel(
      body,
      out_type=jax.ShapeDtypeStruct((NC, N, Q), jnp.float32),
      mesh=_mesh,
      compiler_params=pltpu.CompilerParams(use_tc_tiling_on_sc=False),
      scratch_types=[
          pltpu.VMEM_SHARED((N, Q), jnp.float32),
          pltpu.VMEM((RSUB, WSUB), jnp.int32),
          pltpu.VMEM((WSUB, Q), jnp.float32),
          pltpu.VMEM((ZROWS, Q), jnp.float32),
      ],
  )(dst_rs)


def _sc_agg16(src_rs, dst_rs, tab):
  ept = E // NC // NS
  nch = ept // CHUNK

  def body(src_rs, dst_rs, tab, outp, acc, gidx, didx, rows, sem):
    c = lax.axis_index("c")
    s = lax.axis_index("s")
    _zero_fill(rows, CHUNK)
    _zero_acc(acc, rows.at[pl.ds(0, ZROWS)], s)
    plsc.subcore_barrier()
    row0 = (c * (E // NC) + s * ept) // WSUB
    def chunk(ch, carry):
      r0 = row0 + ch * (CHUNK // WSUB)
      pltpu.sync_copy(src_rs.at[pl.ds(r0, RSUB)], gidx)
      pltpu.sync_copy(dst_rs.at[pl.ds(r0, RSUB)], didx)
      cps = [
          pltpu.async_copy(tab.at[gidx.at[r]],
                           rows.at[pl.ds(r * WSUB, WSUB)], sem)
          for r in range(RSUB)
      ]
      for cp in cps:
        cp.wait()
      for r in range(RSUB):
        pltpu.sync_copy(rows.at[pl.ds(r * WSUB, WSUB)],
                        acc.at[didx.at[r]], add=True)
      return carry
    lax.fori_loop(0, nch, chunk, None)
    plsc.subcore_barrier()
    _dump_acc(acc, outp, c, s)

  return pl.kernel(
      body,
      out_type=jax.ShapeDtypeStruct((NC, N, Q), jnp.float32),
      mesh=_mesh,
      compiler_params=pltpu.CompilerParams(use_tc_tiling_on_sc=False),
      scratch_types=[
          pltpu.VMEM_SHARED((N, Q), jnp.float32),
          pltpu.VMEM((RSUB, WSUB), jnp.int32),
          pltpu.VMEM((RSUB, WSUB), jnp.int32),
          pltpu.VMEM((CHUNK, Q), jnp.float32),
          pltpu.SemaphoreType.DMA,
      ],
  )(src_rs, dst_rs, tab)


def _sc_agg64(src_rs, dst_rs, v0, v1, v2, v3):
  ept = E // NS
  nch = ept // CHUNK

  def body(src_rs, dst_rs, v0, v1, v2, v3, outp,
           acc, gidx, didx, rows, sem):
    c = lax.axis_index("c")
    s = lax.axis_index("s")
    row0 = s * ept // WSUB

    def edge_pass(tab):
      def chunk(ch, carry):
        r0 = row0 + ch * (CHUNK // WSUB)
        pltpu.sync_copy(src_rs.at[pl.ds(r0, RSUB)], gidx)
        pltpu.sync_copy(dst_rs.at[pl.ds(r0, RSUB)], didx)
        cps = [
            pltpu.async_copy(tab.at[gidx.at[r]],
                             rows.at[pl.ds(r * WSUB, WSUB)], sem)
            for r in range(RSUB)
        ]
        for cp in cps:
          cp.wait()
        for r in range(RSUB):
          pltpu.sync_copy(rows.at[pl.ds(r * WSUB, WSUB)],
                          acc.at[didx.at[r]], add=True)
        return carry
      lax.fori_loop(0, nch, chunk, None)

    for rnd in range(2):
      _zero_fill(rows, CHUNK)
      _zero_acc(acc, rows.at[pl.ds(0, ZROWS)], s)
      plsc.subcore_barrier()
      ta = v0 if rnd == 0 else v2
      tb = v1 if rnd == 0 else v3
      @pl.when(c == 0)
      def _():
        edge_pass(ta)
      @pl.when(c == 1)
      def _():
        edge_pass(tb)
      plsc.subcore_barrier()
      _dump_acc(acc, outp, 2 * rnd + c, s)
      plsc.subcore_barrier()

  return pl.kernel(
      body,
      out_type=jax.ShapeDtypeStruct((NQ, N, Q), jnp.float32),
      mesh=_mesh,
      compiler_params=pltpu.CompilerParams(use_tc_tiling_on_sc=False),
      scratch_types=[
          pltpu.VMEM_SHARED((N, Q), jnp.float32),
          pltpu.VMEM((RSUB, WSUB), jnp.int32),
          pltpu.VMEM((RSUB, WSUB), jnp.int32),
          pltpu.VMEM((CHUNK, Q), jnp.float32),
          pltpu.SemaphoreType.DMA,
      ],
  )(src_rs, dst_rs, v0, v1, v2, v3)


def _sc_pool(h2p, dep_p, bat_p):
  nch = NODES_PER_TILE // PCHUNK

  def body(h2p, dep_p, bat_p, sums_o, maxs_o, cnts_o,
           sums, maxs, cnts, hbuf, dbuf, bbuf):
    c = lax.axis_index("c")
    s = lax.axis_index("s")
    wid = s * NC + c
    e0 = (1 - jnp.minimum(lax.iota(jnp.int32, Q), 1)).astype(jnp.float32)
    def za(i, carry):
      for j in range(4):
        sums[i, pl.ds(16 * j, 16)] = jnp.zeros((16,), jnp.float32)
        maxs[i, pl.ds(16 * j, 16)] = jnp.zeros((16,), jnp.float32)
      cnts[i, :] = jnp.zeros((16,), jnp.float32)
      return carry
    lax.fori_loop(0, SEGP, za, None)

    def chunk(ch, carry):
      base = wid * NODES_PER_TILE + ch * PCHUNK
      pltpu.sync_copy(h2p.at[pl.ds(base, PCHUNK)], hbuf)
      pltpu.sync_copy(dep_p.at[pl.ds(base, PCHUNK)], dbuf.at[pl.ds(0, PCHUNK)])
      pltpu.sync_copy(bat_p.at[pl.ds(base, PCHUNK)], bbuf.at[pl.ds(0, PCHUNK)])
      def node(i, carry2):
        d = dbuf[pl.ds(i, 16)][0]
        b = bbuf[pl.ds(i, 16)][0]
        sgi = b * LVL + jnp.clip(d, 0, LVL - 1)
        for j in range(4):
          hv = hbuf[i, pl.ds(16 * j, 16)]
          sums[sgi, pl.ds(16 * j, 16)] = sums[sgi, pl.ds(16 * j, 16)] + hv
          maxs[sgi, pl.ds(16 * j, 16)] = jnp.maximum(
              maxs[sgi, pl.ds(16 * j, 16)], hv)
        cnts[sgi, :] = cnts[sgi, :] + e0
        return carry2
      lax.fori_loop(0, PCHUNK, node, None)
      return carry
    lax.fori_loop(0, nch, chunk, None)

    pltpu.sync_copy(sums.at[pl.ds(0, SEG)], sums_o.at[wid])
    pltpu.sync_copy(maxs.at[pl.ds(0, SEG)], maxs_o.at[wid])
    pltpu.sync_copy(cnts.at[pl.ds(0, SEG)], cnts_o.at[wid])

  return pl.kernel(
      body,
      out_type=[
          jax.ShapeDtypeStruct((NW, SEG, H), jnp.float32),
          jax.ShapeDtypeStruct((NW, SEG, H), jnp.float32),
          jax.ShapeDtypeStruct((NW, SEG, Q), jnp.float32),
      ],
      mesh=_mesh,
      compiler_params=pltpu.CompilerParams(use_tc_tiling_on_sc=False),
      scratch_types=[
          pltpu.VMEM((SEGP, H), jnp.float32),
          pltpu.VMEM((SEGP, H), jnp.float32),
          pltpu.VMEM((SEGP, Q), jnp.float32),
          pltpu.VMEM((PCHUNK, H), jnp.float32),
          pltpu.VMEM((PCHUNK + 16,), jnp.int32),
          pltpu.VMEM((PCHUNK + 16,), jnp.int32),
      ],
  )(h2p, dep_p, bat_p)


_TBLK = 2000


def _tc_u16(degp, x):
  def bodyfn(degp_ref, x_ref, o_ref):
    deg = degp_ref[0, :, 0:1] + degp_ref[1, :, 0:1] + 1.0
    dinv = lax.rsqrt(deg)
    u = dinv * x_ref[...]
    o_ref[...] = jnp.concatenate(
        [u, dinv, jnp.zeros((_TBLK, Q - 3), jnp.float32)], axis=1)
  return pl.pallas_call(
      bodyfn,
      grid=(N // _TBLK,),
      in_specs=[
          pl.BlockSpec((NC, _TBLK, Q), lambda i: (0, i, 0)),
          pl.BlockSpec((_TBLK, IN_DIM), lambda i: (i, 0)),
      ],
      out_specs=pl.BlockSpec((_TBLK, Q), lambda i: (i, 0)),
      out_shape=jax.ShapeDtypeStruct((N, Q), jnp.float32),
  )(degp, x)


def _tc_h1v(aggp, u16, W1, b1):
  def bodyfn(aggp_ref, u16_ref, w1_ref, b1_ref, o0, o1, o2, o3):
    su = (aggp_ref[0, :, 0:IN_DIM] + aggp_ref[1, :, 0:IN_DIM]
          + u16_ref[:, 0:IN_DIM])
    xw = jnp.dot(su, w1_ref[...], preferred_element_type=jnp.float32)
    dinv = u16_ref[:, 2:3]
    h1 = jnp.maximum(dinv * xw + b1_ref[...], 0.0)
    v = dinv * h1
    o0[...] = v[:, 0:16]
    o1[...] = v[:, 16:32]
    o2[...] = v[:, 32:48]
    o3[...] = v[:, 48:64]
  qspec = pl.BlockSpec((_TBLK, Q), lambda i: (i, 0))
  return pl.pallas_call(
      bodyfn,
      grid=(N // _TBLK,),
      in_specs=[
          pl.BlockSpec((NC, _TBLK, Q), lambda i: (0, i, 0)),
          pl.BlockSpec((_TBLK, Q), lambda i: (i, 0)),
          pl.BlockSpec((IN_DIM, H), lambda i: (0, 0)),
          pl.BlockSpec((1, H), lambda i: (0, 0)),
      ],
      out_specs=[qspec, qspec, qspec, qspec],
      out_shape=[jax.ShapeDtypeStruct((N, Q), jnp.float32)] * 4,
  )(aggp, u16, W1, b1)


def _tc_h2(aq, v0, v1, v2, v3, u16, W2, b2):
  def bodyfn(aq_ref, v0r, v1r, v2r, v3r, u16_ref, w2_ref, b2_ref, o_ref):
    g = jnp.concatenate(
        [aq_ref[0] + v0r[...], aq_ref[1] + v1r[...],
         aq_ref[2] + v2r[...], aq_ref[3] + v3r[...]], axis=1)
    dinv = u16_ref[:, 2:3]
    hw = jnp.dot(g, w2_ref[...], preferred_element_type=jnp.float32)
    o_ref[...] = jnp.maximum(dinv * hw + b2_ref[...], 0.0)
  qspec = pl.BlockSpec((_TBLK, Q), lambda i: (i, 0))
  return pl.pallas_call(
      bodyfn,
      grid=(N // _TBLK,),
      in_specs=[
          pl.BlockSpec((NQ, _TBLK, Q), lambda i: (0, i, 0)),
          qspec, qspec, qspec, qspec,
          qspec,
          pl.BlockSpec((H, H), lambda i: (0, 0)),
          pl.BlockSpec((1, H), lambda i: (0, 0)),
      ],
      out_specs=pl.BlockSpec((_TBLK, H), lambda i: (i, 0)),
      out_shape=jax.ShapeDtypeStruct((N, H), jnp.float32),
  )(aq, v0, v1, v2, v3, u16, W2, b2)


def _tc_head(sums_p, maxs_p, cnts_p, cwt, cb, f1w, f1b, f2w, f2b):
  def bodyfn(sums_ref, maxs_ref, cnts_ref, cwt_ref, cb_ref,
             f1w_ref, f1b_ref, f2w_ref, f2b_ref, o_ref):
    sums = jnp.sum(sums_ref[...], axis=0)
    maxs = jnp.max(maxs_ref[...], axis=0)
    cnts = jnp.sum(cnts_ref[:, :, 0], axis=0)
    means = sums / jnp.maximum(cnts, 1.0)[:, None]
    feats = jnp.concatenate([means, maxs], axis=1)
    logits = []
    for b in range(NB):
      M = feats[b * LVL:(b + 1) * LVL, :]
      Su = jnp.concatenate(
          [jnp.zeros((1, 2 * H), jnp.float32), M[:-1]], axis=0)
      Sd = jnp.concatenate(
          [M[1:], jnp.zeros((1, 2 * H), jnp.float32)], axis=0)
      conv = (jnp.dot(Su, cwt_ref[0], preferred_element_type=jnp.float32)
              + jnp.dot(M, cwt_ref[1], preferred_element_type=jnp.float32)
              + jnp.dot(Sd, cwt_ref[2], preferred_element_type=jnp.float32))
      conv = jnp.maximum(conv + cb_ref[...], 0.0)
      pooled = jnp.mean(conv, axis=0, keepdims=True)
      z = jnp.maximum(
          jnp.dot(pooled, f1w_ref[...], preferred_element_type=jnp.float32)
          + f1b_ref[...], 0.0)
      logits.append(
          jnp.dot(z, f2w_ref[...], preferred_element_type=jnp.float32)
          + f2b_ref[...])
    o_ref[...] = jax.nn.sigmoid(jnp.concatenate(logits, axis=1))
  return pl.pallas_call(
      bodyfn,
      out_shape=jax.ShapeDtypeStruct((1, NB), jnp.float32),
  )(sums_p, maxs_p, cnts_p, cwt, cb, f1w, f1b, f2w, f2b)


def kernel(x, edge_index, node_depth, batch_index, W1, b1, W2, b2,
           conv_w, conv_b, fc1_w, fc1_b, fc2_w, fc2_b):
  src_rs = edge_index[0].reshape(E // WSUB, WSUB)
  dst_rs = edge_index[1].reshape(E // WSUB, WSUB)

  degp = _sc_deg(dst_rs)
  u16 = _tc_u16(degp, x)
  aggp = _sc_agg16(src_rs, dst_rs, u16)
  v0, v1, v2, v3 = _tc_h1v(aggp, u16, W1, b1.reshape(1, H))
  aq = _sc_agg64(src_rs, dst_rs, v0, v1, v2, v3)
  h2 = _tc_h2(aq, v0, v1, v2, v3, u16, W2, b2.reshape(1, H))

  h2p = jnp.zeros((NP, H), jnp.float32).at[:N].set(h2)
  dep_p = jnp.zeros((NP,), jnp.int32).at[:N].set(node_depth)
  bat_p = jnp.full((NP,), NB, jnp.int32).at[:N].set(batch_index)

  sums_p, maxs_p, cnts_p = _sc_pool(h2p, dep_p, bat_p)

  out = _tc_head(sums_p, maxs_p, cnts_p,
                 conv_w.transpose(2, 1, 0), conv_b.reshape(1, CC),
                 fc1_w, fc1_b.reshape(1, CC), fc2_w, fc2_b.reshape(1, 1))
  return out.reshape(-1)


# speedup vs baseline: 24.6282x; 1.2648x over previous
import jax
import jax.numpy as jnp
from jax import lax
from jax.experimental import pallas as pl
from jax.experimental.pallas import tpu as pltpu
from jax.experimental.pallas import tpu_sc as plsc

N = 100000
E = 1600000
IN_DIM = 2
H = 64
CC = 64
LVL = 128
NB = 4
SEG = NB * LVL
SEGP = 528

NC = 2
NS = 16
NW = NC * NS

Q = 16
NQ = 4

WSUB = 125
RSUB = 5
CHUNK = RSUB * WSUB

ROWS_PER_TILE = N // NS
ZROWS = 625

NP = 102400
NODES_PER_TILE = NP // NW
PCHUNK = 640

_mesh = plsc.VectorSubcoreMesh(core_axis_name="c", subcore_axis_name="s")


def _zero_fill(buf, nrows):
  def zb(i, carry):
    buf[i, :] = jnp.zeros((Q,), jnp.float32)
    return carry
  lax.fori_loop(0, nrows, zb, None)


def _zero_acc(acc, zbuf, s):
  for p in range(ROWS_PER_TILE // ZROWS):
    pltpu.sync_copy(zbuf, acc.at[pl.ds(s * ROWS_PER_TILE + p * ZROWS, ZROWS)])


def _dump_acc(acc, outp, q, s):
  for p in range(ROWS_PER_TILE // ZROWS):
    off = s * ROWS_PER_TILE + p * ZROWS
    pltpu.sync_copy(acc.at[pl.ds(off, ZROWS)], outp.at[q, pl.ds(off, ZROWS)])


def _edge_pipeline(nch, row0, src_rs, dst_rs, tab, acc, idxr, rows,
                   sem_st, sem_g, sem_sc, drain_hbm):
  def stage(ch, slot):
    r0 = row0 + ch * RSUB
    pltpu.async_copy(src_rs.at[pl.ds(r0, RSUB)], idxr.at[slot, 0], sem_st)
    pltpu.async_copy(dst_rs.at[pl.ds(r0, RSUB)], idxr.at[slot, 1], sem_st)

  stage(0, 0)

  def body(ch, carry):
    slot = lax.rem(ch, 3)
    p = lax.rem(ch, 2)
    @pl.when(ch >= 2)
    def _():
      pltpu.make_async_copy(drain_hbm, rows.at[0], sem_sc).wait()
    pltpu.make_async_copy(src_rs.at[pl.ds(0, RSUB)], idxr.at[0, 0], sem_st).wait()
    pltpu.make_async_copy(src_rs.at[pl.ds(0, RSUB)], idxr.at[0, 0], sem_st).wait()
    for r in range(RSUB):
      pltpu.async_copy(tab.at[idxr.at[slot, 0, r]],
                       rows.at[p, pl.ds(r * WSUB, WSUB)], sem_g)
    @pl.when(ch + 1 < nch)
    def _():
      stage(ch + 1, lax.rem(ch + 1, 3))
    pltpu.make_async_copy(drain_hbm, rows.at[0], sem_g).wait()
    for r in range(RSUB):
      pltpu.async_copy(rows.at[p, pl.ds(r * WSUB, WSUB)],
                       acc.at[idxr.at[slot, 1, r]], sem_sc, add=True)
    return carry

  lax.fori_loop(0, nch, body, None)
  pltpu.make_async_copy(drain_hbm, rows.at[0], sem_sc).wait()
  pltpu.make_async_copy(drain_hbm, rows.at[0], sem_sc).wait()


def _sc_deg(dst_rs):
  ept = E // NC // NS
  nch = ept // CHUNK

  def body(dst_rs, outp, acc, didxr, ones_rows, zbuf, sem_st, sem_sc):
    c = lax.axis_index("c")
    s = lax.axis_index("s")
    _zero_fill(zbuf, ZROWS)
    def ob(i, carry):
      ones_rows[i, :] = (1 - jnp.minimum(lax.iota(jnp.int32, Q), 1)).astype(jnp.float32)
      return carry
    lax.fori_loop(0, WSUB, ob, None)
    _zero_acc(acc, zbuf, s)
    plsc.subcore_barrier()
    row0 = (c * (E // NC) + s * ept) // WSUB
    drain_hbm = outp.at[0, pl.ds(0, CHUNK)]

    def stage(ch, slot):
      pltpu.async_copy(dst_rs.at[pl.ds(row0 + ch * RSUB, RSUB)],
                       didxr.at[slot], sem_st)
    stage(0, 0)

    def chunk(ch, carry):
      slot = lax.rem(ch, 3)
      @pl.when(ch >= 2)
      def _():
        pltpu.make_async_copy(drain_hbm, zbuf, sem_sc).wait()
      pltpu.make_async_copy(dst_rs.at[pl.ds(0, RSUB)], didxr.at[0], sem_st).wait()
      @pl.when(ch + 1 < nch)
      def _():
        stage(ch + 1, lax.rem(ch + 1, 3))
      for r in range(RSUB):
        pltpu.async_copy(ones_rows, acc.at[didxr.at[slot, r]], sem_sc, add=True)
      return carry
    lax.fori_loop(0, nch, chunk, None)
    pltpu.make_async_copy(drain_hbm, zbuf, sem_sc).wait()
    pltpu.make_async_copy(drain_hbm, zbuf, sem_sc).wait()
    plsc.subcore_barrier()
    _dump_acc(acc, outp, c, s)

  return pl.kernel(
      body,
      out_type=jax.ShapeDtypeStruct((NC, N, Q), jnp.float32),
      mesh=_mesh,
      compiler_params=pltpu.CompilerParams(use_tc_tiling_on_sc=False),
      scratch_types=[
          pltpu.VMEM_SHARED((N, Q), jnp.float32),
          pltpu.VMEM((3, RSUB, WSUB), jnp.int32),
          pltpu.VMEM((WSUB, Q), jnp.float32),
          pltpu.VMEM((ZROWS, Q), jnp.float32),
          pltpu.SemaphoreType.DMA,
          pltpu.SemaphoreType.DMA,
      ],
  )(dst_rs)


def _sc_agg16(src_rs, dst_rs, tab):
  ept = E // NC // NS
  nch = ept // CHUNK

  def body(src_rs, dst_rs, tab, outp, acc, idxr, rows,
           sem_st, sem_g, sem_sc):
    c = lax.axis_index("c")
    s = lax.axis_index("s")
    _zero_fill(rows.at[0], CHUNK)
    _zero_acc(acc, rows.at[0, pl.ds(0, ZROWS)], s)
    plsc.subcore_barrier()
    row0 = (c * (E // NC) + s * ept) // WSUB
    _edge_pipeline(nch, row0, src_rs, dst_rs, tab, acc, idxr, rows,
                   sem_st, sem_g, sem_sc, tab.at[pl.ds(0, CHUNK)])
    plsc.subcore_barrier()
    _dump_acc(acc, outp, c, s)

  return pl.kernel(
      body,
      out_type=jax.ShapeDtypeStruct((NC, N, Q), jnp.float32),
      mesh=_mesh,
      compiler_params=pltpu.CompilerParams(use_tc_tiling_on_sc=False),
      scratch_types=[
          pltpu.VMEM_SHARED((N, Q), jnp.float32),
          pltpu.VMEM((3, 2, RSUB, WSUB), jnp.int32),
          pltpu.VMEM((2, CHUNK, Q), jnp.float32),
          pltpu.SemaphoreType.DMA,
          pltpu.SemaphoreType.DMA,
          pltpu.SemaphoreType.DMA,
      ],
  )(src_rs, dst_rs, tab)


def _sc_agg64(src_rs, dst_rs, v0, v1, v2, v3):
  ept = E // NS
  nch = ept // CHUNK

  def body(src_rs, dst_rs, v0, v1, v2, v3, outp,
           acc, idxr, rows, sem_st, sem_g, sem_sc):
    c = lax.axis_index("c")
    s = lax.axis_index("s")
    row0 = s * ept // WSUB

    for rnd in range(2):
      _zero_fill(rows.at[0], CHUNK)
      _zero_acc(acc, rows.at[0, pl.ds(0, ZROWS)], s)
      plsc.subcore_barrier()
      ta = v0 if rnd == 0 else v2
      tb = v1 if rnd == 0 else v3
      @pl.when(c == 0)
      def _():
        _edge_pipeline(nch, row0, src_rs, dst_rs, ta, acc, idxr, rows,
                       sem_st, sem_g, sem_sc, ta.at[pl.ds(0, CHUNK)])
      @pl.when(c == 1)
      def _():
        _edge_pipeline(nch, row0, src_rs, dst_rs, tb, acc, idxr, rows,
                       sem_st, sem_g, sem_sc, tb.at[pl.ds(0, CHUNK)])
      plsc.subcore_barrier()
      _dump_acc(acc, outp, 2 * rnd + c, s)
      plsc.subcore_barrier()

  return pl.kernel(
      body,
      out_type=jax.ShapeDtypeStruct((NQ, N, Q), jnp.float32),
      mesh=_mesh,
      compiler_params=pltpu.CompilerParams(use_tc_tiling_on_sc=False),
      scratch_types=[
          pltpu.VMEM_SHARED((N, Q), jnp.float32),
          pltpu.VMEM((3, 2, RSUB, WSUB), jnp.int32),
          pltpu.VMEM((2, CHUNK, Q), jnp.float32),
          pltpu.SemaphoreType.DMA,
          pltpu.SemaphoreType.DMA,
          pltpu.SemaphoreType.DMA,
      ],
  )(src_rs, dst_rs, v0, v1, v2, v3)


def _sc_pool(h2p, dep_p, bat_p):
  nch = NODES_PER_TILE // PCHUNK

  def body(h2p, dep_p, bat_p, sums_o, maxs_o, cnts_o,
           sums, maxs, cnts, hbuf, dbuf, bbuf):
    c = lax.axis_index("c")
    s = lax.axis_index("s")
    wid = s * NC + c
    e0 = (1 - jnp.minimum(lax.iota(jnp.int32, Q), 1)).astype(jnp.float32)
    def za(i, carry):
      for j in range(4):
        sums[i, pl.ds(16 * j, 16)] = jnp.zeros((16,), jnp.float32)
        maxs[i, pl.ds(16 * j, 16)] = jnp.zeros((16,), jnp.float32)
      cnts[i, :] = jnp.zeros((16,), jnp.float32)
      return carry
    lax.fori_loop(0, SEGP, za, None)

    def chunk(ch, carry):
      base = wid * NODES_PER_TILE + ch * PCHUNK
      pltpu.sync_copy(h2p.at[pl.ds(base, PCHUNK)], hbuf)
      pltpu.sync_copy(dep_p.at[pl.ds(base, PCHUNK)], dbuf.at[pl.ds(0, PCHUNK)])
      pltpu.sync_copy(bat_p.at[pl.ds(base, PCHUNK)], bbuf.at[pl.ds(0, PCHUNK)])
      def node(i, carry2):
        d = dbuf[pl.ds(i, 16)][0]
        b = bbuf[pl.ds(i, 16)][0]
        sgi = b * LVL + jnp.clip(d, 0, LVL - 1)
        for j in range(4):
          hv = hbuf[i, pl.ds(16 * j, 16)]
          sums[sgi, pl.ds(16 * j, 16)] = sums[sgi, pl.ds(16 * j, 16)] + hv
          maxs[sgi, pl.ds(16 * j, 16)] = jnp.maximum(
              maxs[sgi, pl.ds(16 * j, 16)], hv)
        cnts[sgi, :] = cnts[sgi, :] + e0
        return carry2
      lax.fori_loop(0, PCHUNK, node, None)
      return carry
    lax.fori_loop(0, nch, chunk, None)

    pltpu.sync_copy(sums.at[pl.ds(0, SEG)], sums_o.at[wid])
    pltpu.sync_copy(maxs.at[pl.ds(0, SEG)], maxs_o.at[wid])
    pltpu.sync_copy(cnts.at[pl.ds(0, SEG)], cnts_o.at[wid])

  return pl.kernel(
      body,
      out_type=[
          jax.ShapeDtypeStruct((NW, SEG, H), jnp.float32),
          jax.ShapeDtypeStruct((NW, SEG, H), jnp.float32),
          jax.ShapeDtypeStruct((NW, SEG, Q), jnp.float32),
      ],
      mesh=_mesh,
      compiler_params=pltpu.CompilerParams(use_tc_tiling_on_sc=False),
      scratch_types=[
          pltpu.VMEM((SEGP, H), jnp.float32),
          pltpu.VMEM((SEGP, H), jnp.float32),
          pltpu.VMEM((SEGP, Q), jnp.float32),
          pltpu.VMEM((PCHUNK, H), jnp.float32),
          pltpu.VMEM((PCHUNK + 16,), jnp.int32),
          pltpu.VMEM((PCHUNK + 16,), jnp.int32),
      ],
  )(h2p, dep_p, bat_p)


_TBLK = 2000


def _tc_u16(degp, x):
  def bodyfn(degp_ref, x_ref, o_ref):
    deg = degp_ref[0, :, 0:1] + degp_ref[1, :, 0:1] + 1.0
    dinv = lax.rsqrt(deg)
    u = dinv * x_ref[...]
    o_ref[...] = jnp.concatenate(
        [u, dinv, jnp.zeros((_TBLK, Q - 3), jnp.float32)], axis=1)
  return pl.pallas_call(
      bodyfn,
      grid=(N // _TBLK,),
      in_specs=[
          pl.BlockSpec((NC, _TBLK, Q), lambda i: (0, i, 0)),
          pl.BlockSpec((_TBLK, IN_DIM), lambda i: (i, 0)),
      ],
      out_specs=pl.BlockSpec((_TBLK, Q), lambda i: (i, 0)),
      out_shape=jax.ShapeDtypeStruct((N, Q), jnp.float32),
  )(degp, x)


def _tc_h1v(aggp, u16, W1, b1):
  def bodyfn(aggp_ref, u16_ref, w1_ref, b1_ref, o0, o1, o2, o3):
    su = (aggp_ref[0, :, 0:IN_DIM] + aggp_ref[1, :, 0:IN_DIM]
          + u16_ref[:, 0:IN_DIM])
    xw = jnp.dot(su, w1_ref[...], preferred_element_type=jnp.float32)
    dinv = u16_ref[:, 2:3]
    h1 = jnp.maximum(dinv * xw + b1_ref[...], 0.0)
    v = dinv * h1
    o0[...] = v[:, 0:16]
    o1[...] = v[:, 16:32]
    o2[...] = v[:, 32:48]
    o3[...] = v[:, 48:64]
  qspec = pl.BlockSpec((_TBLK, Q), lambda i: (i, 0))
  return pl.pallas_call(
      bodyfn,
      grid=(N // _TBLK,),
      in_specs=[
          pl.BlockSpec((NC, _TBLK, Q), lambda i: (0, i, 0)),
          pl.BlockSpec((_TBLK, Q), lambda i: (i, 0)),
          pl.BlockSpec((IN_DIM, H), lambda i: (0, 0)),
          pl.BlockSpec((1, H), lambda i: (0, 0)),
      ],
      out_specs=[qspec, qspec, qspec, qspec],
      out_shape=[jax.ShapeDtypeStruct((N, Q), jnp.float32)] * 4,
  )(aggp, u16, W1, b1)


def _tc_h2(aq, v0, v1, v2, v3, u16, W2, b2):
  def bodyfn(aq_ref, v0r, v1r, v2r, v3r, u16_ref, w2_ref, b2_ref, o_ref):
    g = jnp.concatenate(
        [aq_ref[0] + v0r[...], aq_ref[1] + v1r[...],
         aq_ref[2] + v2r[...], aq_ref[3] + v3r[...]], axis=1)
    dinv = u16_ref[:, 2:3]
    hw = jnp.dot(g, w2_ref[...], preferred_element_type=jnp.float32)
    o_ref[...] = jnp.maximum(dinv * hw + b2_ref[...], 0.0)
  qspec = pl.BlockSpec((_TBLK, Q), lambda i: (i, 0))
  return pl.pallas_call(
      bodyfn,
      grid=(N // _TBLK,),
      in_specs=[
          pl.BlockSpec((NQ, _TBLK, Q), lambda i: (0, i, 0)),
          qspec, qspec, qspec, qspec,
          qspec,
          pl.BlockSpec((H, H), lambda i: (0, 0)),
          pl.BlockSpec((1, H), lambda i: (0, 0)),
      ],
      out_specs=pl.BlockSpec((_TBLK, H), lambda i: (i, 0)),
      out_shape=jax.ShapeDtypeStruct((N, H), jnp.float32),
  )(aq, v0, v1, v2, v3, u16, W2, b2)


def _tc_head(sums_p, maxs_p, cnts_p, cwt, cb, f1w, f1b, f2w, f2b):
  def bodyfn(sums_ref, maxs_ref, cnts_ref, cwt_ref, cb_ref,
             f1w_ref, f1b_ref, f2w_ref, f2b_ref, o_ref):
    sums = jnp.sum(sums_ref[...], axis=0)
    maxs = jnp.max(maxs_ref[...], axis=0)
    cnts = jnp.sum(cnts_ref[:, :, 0], axis=0)
    means = sums / jnp.maximum(cnts, 1.0)[:, None]
    feats = jnp.concatenate([means, maxs], axis=1)
    logits = []
    for b in range(NB):
      M = feats[b * LVL:(b + 1) * LVL, :]
      Su = jnp.concatenate(
          [jnp.zeros((1, 2 * H), jnp.float32), M[:-1]], axis=0)
      Sd = jnp.concatenate(
          [M[1:], jnp.zeros((1, 2 * H), jnp.float32)], axis=0)
      conv = (jnp.dot(Su, cwt_ref[0], preferred_element_type=jnp.float32)
              + jnp.dot(M, cwt_ref[1], preferred_element_type=jnp.float32)
              + jnp.dot(Sd, cwt_ref[2], preferred_element_type=jnp.float32))
      conv = jnp.maximum(conv + cb_ref[...], 0.0)
      pooled = jnp.mean(conv, axis=0, keepdims=True)
      z = jnp.maximum(
          jnp.dot(pooled, f1w_ref[...], preferred_element_type=jnp.float32)
          + f1b_ref[...], 0.0)
      logits.append(
          jnp.dot(z, f2w_ref[...], preferred_element_type=jnp.float32)
          + f2b_ref[...])
    o_ref[...] = jax.nn.sigmoid(jnp.concatenate(logits, axis=1))
  return pl.pallas_call(
      bodyfn,
      out_shape=jax.ShapeDtypeStruct((1, NB), jnp.float32),
  )(sums_p, maxs_p, cnts_p, cwt, cb, f1w, f1b, f2w, f2b)


def kernel(x, edge_index, node_depth, batch_index, W1, b1, W2, b2,
           conv_w, conv_b, fc1_w, fc1_b, fc2_w, fc2_b):
  src_rs = edge_index[0].reshape(E // WSUB, WSUB)
  dst_rs = edge_index[1].reshape(E // WSUB, WSUB)

  degp = _sc_deg(dst_rs)
  u16 = _tc_u16(degp, x)
  aggp = _sc_agg16(src_rs, dst_rs, u16)
  v0, v1, v2, v3 = _tc_h1v(aggp, u16, W1, b1.reshape(1, H))
  aq = _sc_agg64(src_rs, dst_rs, v0, v1, v2, v3)
  h2 = _tc_h2(aq, v0, v1, v2, v3, u16, W2, b2.reshape(1, H))

  h2p = jnp.zeros((NP, H), jnp.float32).at[:N].set(h2)
  dep_p = jnp.zeros((NP,), jnp.int32).at[:N].set(node_depth)
  bat_p = jnp.full((NP,), NB, jnp.int32).at[:N].set(batch_index)

  sums_p, maxs_p, cnts_p = _sc_pool(h2p, dep_p, bat_p)

  out = _tc_head(sums_p, maxs_p, cnts_p,
                 conv_w.transpose(2, 1, 0), conv_b.reshape(1, CC),
                 fc1_w, fc1_b.reshape(1, CC), fc2_w, fc2_b.reshape(1, 1))
  return out.reshape(-1)


# speedup vs baseline: 25.0545x; 1.0173x over previous
import jax
import jax.numpy as jnp
from jax import lax
from jax.experimental import pallas as pl
from jax.experimental.pallas import tpu as pltpu
from jax.experimental.pallas import tpu_sc as plsc

N = 100000
E = 1600000
IN_DIM = 2
H = 64
CC = 64
LVL = 128
NB = 4
SEG = NB * LVL
SEGP = 528

NC = 2
NS = 16
NW = NC * NS

Q = 16
NQ = 4

WSUB = 125
RSUB = 5
CHUNK = RSUB * WSUB

ROWS_PER_TILE = N // NS
ZROWS = 625

NP = 102400
NODES_PER_TILE = NP // NW
PCHUNK = 640

_mesh = plsc.VectorSubcoreMesh(core_axis_name="c", subcore_axis_name="s")


def _zero_fill(buf, nrows):
  def zb(i, carry):
    buf[i, :] = jnp.zeros((Q,), jnp.float32)
    return carry
  lax.fori_loop(0, nrows, zb, None)


def _zero_acc(acc, zbuf, s):
  for p in range(ROWS_PER_TILE // ZROWS):
    pltpu.sync_copy(zbuf, acc.at[pl.ds(s * ROWS_PER_TILE + p * ZROWS, ZROWS)])


def _dump_acc(acc, outp, q, s):
  for p in range(ROWS_PER_TILE // ZROWS):
    off = s * ROWS_PER_TILE + p * ZROWS
    pltpu.sync_copy(acc.at[pl.ds(off, ZROWS)], outp.at[q, pl.ds(off, ZROWS)])


def _edge_pipeline(nch, row0, src_rs, dst_rs, tab, acc, idxr, rows,
                   sem_st, sem_g, sem_sc, drain_hbm):
  def stage(ch, slot):
    r0 = row0 + ch * RSUB
    pltpu.async_copy(src_rs.at[pl.ds(r0, RSUB)], idxr.at[slot, 0], sem_st)
    pltpu.async_copy(dst_rs.at[pl.ds(r0, RSUB)], idxr.at[slot, 1], sem_st)

  stage(0, 0)

  def body(ch, carry):
    slot = lax.rem(ch, 3)
    p = lax.rem(ch, 2)
    @pl.when(ch >= 2)
    def _():
      pltpu.make_async_copy(drain_hbm, rows.at[0], sem_sc).wait()
    pltpu.make_async_copy(src_rs.at[pl.ds(0, RSUB)], idxr.at[0, 0], sem_st).wait()
    pltpu.make_async_copy(src_rs.at[pl.ds(0, RSUB)], idxr.at[0, 0], sem_st).wait()
    for r in range(RSUB):
      pltpu.async_copy(tab.at[idxr.at[slot, 0, r]],
                       rows.at[p, pl.ds(r * WSUB, WSUB)], sem_g)
    @pl.when(ch + 1 < nch)
    def _():
      stage(ch + 1, lax.rem(ch + 1, 3))
    pltpu.make_async_copy(drain_hbm, rows.at[0], sem_g).wait()
    for r in range(RSUB):
      pltpu.async_copy(rows.at[p, pl.ds(r * WSUB, WSUB)],
                       acc.at[idxr.at[slot, 1, r]], sem_sc, add=True)
    return carry

  lax.fori_loop(0, nch, body, None)
  pltpu.make_async_copy(drain_hbm, rows.at[0], sem_sc).wait()
  pltpu.make_async_copy(drain_hbm, rows.at[0], sem_sc).wait()


def _sc_deg(dst_rs):
  ept = E // NC // NS
  nch = ept // CHUNK

  def body(dst_rs, outp, acc, didxr, ones_rows, zbuf, sem_st, sem_sc):
    c = lax.axis_index("c")
    s = lax.axis_index("s")
    _zero_fill(zbuf, ZROWS)
    def ob(i, carry):
      ones_rows[i, :] = (1 - jnp.minimum(lax.iota(jnp.int32, Q), 1)).astype(jnp.float32)
      return carry
    lax.fori_loop(0, WSUB, ob, None)
    _zero_acc(acc, zbuf, s)
    plsc.subcore_barrier()
    row0 = (c * (E // NC) + s * ept) // WSUB
    drain_hbm = outp.at[0, pl.ds(0, CHUNK)]

    def stage(ch, slot):
      pltpu.async_copy(dst_rs.at[pl.ds(row0 + ch * RSUB, RSUB)],
                       didxr.at[slot], sem_st)
    stage(0, 0)

    def chunk(ch, carry):
      slot = lax.rem(ch, 3)
      @pl.when(ch >= 2)
      def _():
        pltpu.make_async_copy(drain_hbm, zbuf, sem_sc).wait()
      pltpu.make_async_copy(dst_rs.at[pl.ds(0, RSUB)], didxr.at[0], sem_st).wait()
      @pl.when(ch + 1 < nch)
      def _():
        stage(ch + 1, lax.rem(ch + 1, 3))
      for r in range(RSUB):
        pltpu.async_copy(ones_rows, acc.at[didxr.at[slot, r]], sem_sc, add=True)
      return carry
    lax.fori_loop(0, nch, chunk, None)
    pltpu.make_async_copy(drain_hbm, zbuf, sem_sc).wait()
    pltpu.make_async_copy(drain_hbm, zbuf, sem_sc).wait()
    plsc.subcore_barrier()
    _dump_acc(acc, outp, c, s)

  return pl.kernel(
      body,
      out_type=jax.ShapeDtypeStruct((NC, N, Q), jnp.float32),
      mesh=_mesh,
      compiler_params=pltpu.CompilerParams(use_tc_tiling_on_sc=False),
      scratch_types=[
          pltpu.VMEM_SHARED((N, Q), jnp.float32),
          pltpu.VMEM((3, RSUB, WSUB), jnp.int32),
          pltpu.VMEM((WSUB, Q), jnp.float32),
          pltpu.VMEM((ZROWS, Q), jnp.float32),
          pltpu.SemaphoreType.DMA,
          pltpu.SemaphoreType.DMA,
      ],
  )(dst_rs)


def _sc_agg16(src_rs, dst_rs, tab):
  ept = E // NC // NS
  nch = ept // CHUNK

  def body(src_rs, dst_rs, tab, outp, acc, idxr, rows,
           sem_st, sem_g, sem_sc):
    c = lax.axis_index("c")
    s = lax.axis_index("s")
    _zero_fill(rows.at[0], CHUNK)
    _zero_acc(acc, rows.at[0, pl.ds(0, ZROWS)], s)
    plsc.subcore_barrier()
    row0 = (c * (E // NC) + s * ept) // WSUB
    _edge_pipeline(nch, row0, src_rs, dst_rs, tab, acc, idxr, rows,
                   sem_st, sem_g, sem_sc, tab.at[pl.ds(0, CHUNK)])
    plsc.subcore_barrier()
    _dump_acc(acc, outp, c, s)

  return pl.kernel(
      body,
      out_type=jax.ShapeDtypeStruct((NC, N, Q), jnp.float32),
      mesh=_mesh,
      compiler_params=pltpu.CompilerParams(use_tc_tiling_on_sc=False),
      scratch_types=[
          pltpu.VMEM_SHARED((N, Q), jnp.float32),
          pltpu.VMEM((3, 2, RSUB, WSUB), jnp.int32),
          pltpu.VMEM((2, CHUNK, Q), jnp.float32),
          pltpu.SemaphoreType.DMA,
          pltpu.SemaphoreType.DMA,
          pltpu.SemaphoreType.DMA,
      ],
  )(src_rs, dst_rs, tab)


def _sc_agg64(src_rs, dst_rs, v0, v1, v2, v3):
  ept = E // NS
  nch = ept // CHUNK

  def body(src_rs, dst_rs, v0, v1, v2, v3, outp,
           acc, idxr, rows, sem_st, sem_g, sem_sc):
    c = lax.axis_index("c")
    s = lax.axis_index("s")
    row0 = s * ept // WSUB

    for rnd in range(2):
      _zero_fill(rows.at[0], CHUNK)
      _zero_acc(acc, rows.at[0, pl.ds(0, ZROWS)], s)
      plsc.subcore_barrier()
      ta = v0 if rnd == 0 else v2
      tb = v1 if rnd == 0 else v3
      @pl.when(c == 0)
      def _():
        _edge_pipeline(nch, row0, src_rs, dst_rs, ta, acc, idxr, rows,
                       sem_st, sem_g, sem_sc, ta.at[pl.ds(0, CHUNK)])
      @pl.when(c == 1)
      def _():
        _edge_pipeline(nch, row0, src_rs, dst_rs, tb, acc, idxr, rows,
                       sem_st, sem_g, sem_sc, tb.at[pl.ds(0, CHUNK)])
      plsc.subcore_barrier()
      _dump_acc(acc, outp, 2 * rnd + c, s)
      plsc.subcore_barrier()

  return pl.kernel(
      body,
      out_type=jax.ShapeDtypeStruct((NQ, N, Q), jnp.float32),
      mesh=_mesh,
      compiler_params=pltpu.CompilerParams(use_tc_tiling_on_sc=False),
      scratch_types=[
          pltpu.VMEM_SHARED((N, Q), jnp.float32),
          pltpu.VMEM((3, 2, RSUB, WSUB), jnp.int32),
          pltpu.VMEM((2, CHUNK, Q), jnp.float32),
          pltpu.SemaphoreType.DMA,
          pltpu.SemaphoreType.DMA,
          pltpu.SemaphoreType.DMA,
      ],
  )(src_rs, dst_rs, v0, v1, v2, v3)


def _sc_pool(h2p, dep_p, bat_p):
  nch = NODES_PER_TILE // PCHUNK

  def body(h2p, dep_p, bat_p, sums_o, maxs_o, cnts_o,
           sums, maxs, cnts, hbuf, dbuf, bbuf):
    c = lax.axis_index("c")
    s = lax.axis_index("s")
    wid = s * NC + c
    e0 = (1 - jnp.minimum(lax.iota(jnp.int32, Q), 1)).astype(jnp.float32)
    def za(i, carry):
      for j in range(4):
        sums[i, pl.ds(16 * j, 16)] = jnp.zeros((16,), jnp.float32)
        maxs[i, pl.ds(16 * j, 16)] = jnp.zeros((16,), jnp.float32)
      cnts[i, :] = jnp.zeros((16,), jnp.float32)
      return carry
    lax.fori_loop(0, SEGP, za, None)

    def chunk(ch, carry):
      base = wid * NODES_PER_TILE + ch * PCHUNK
      pltpu.sync_copy(h2p.at[pl.ds(base, PCHUNK)], hbuf)
      pltpu.sync_copy(dep_p.at[pl.ds(base, PCHUNK)], dbuf.at[pl.ds(0, PCHUNK)])
      pltpu.sync_copy(bat_p.at[pl.ds(base, PCHUNK)], bbuf.at[pl.ds(0, PCHUNK)])
      def node(i, carry2):
        d = dbuf[pl.ds(i, 16)][0]
        b = bbuf[pl.ds(i, 16)][0]
        sgi = b * LVL + jnp.clip(d, 0, LVL - 1)
        for j in range(4):
          hv = hbuf[i, pl.ds(16 * j, 16)]
          sums[sgi, pl.ds(16 * j, 16)] = sums[sgi, pl.ds(16 * j, 16)] + hv
          maxs[sgi, pl.ds(16 * j, 16)] = jnp.maximum(
              maxs[sgi, pl.ds(16 * j, 16)], hv)
        cnts[sgi, :] = cnts[sgi, :] + e0
        return carry2
      lax.fori_loop(0, PCHUNK, node, None)
      return carry
    lax.fori_loop(0, nch, chunk, None)

    pltpu.sync_copy(sums.at[pl.ds(0, SEG)], sums_o.at[wid])
    pltpu.sync_copy(maxs.at[pl.ds(0, SEG)], maxs_o.at[wid])
    pltpu.sync_copy(cnts.at[pl.ds(0, SEG)], cnts_o.at[wid])

  return pl.kernel(
      body,
      out_type=[
          jax.ShapeDtypeStruct((NW, SEG, H), jnp.float32),
          jax.ShapeDtypeStruct((NW, SEG, H), jnp.float32),
          jax.ShapeDtypeStruct((NW, SEG, Q), jnp.float32),
      ],
      mesh=_mesh,
      compiler_params=pltpu.CompilerParams(use_tc_tiling_on_sc=False),
      scratch_types=[
          pltpu.VMEM((SEGP, H), jnp.float32),
          pltpu.VMEM((SEGP, H), jnp.float32),
          pltpu.VMEM((SEGP, Q), jnp.float32),
          pltpu.VMEM((PCHUNK, H), jnp.float32),
          pltpu.VMEM((PCHUNK + 16,), jnp.int32),
          pltpu.VMEM((PCHUNK + 16,), jnp.int32),
      ],
  )(h2p, dep_p, bat_p)


_TBLK = 2000


def _tc_u16(degp, x):
  def bodyfn(degp_ref, x_ref, o_ref):
    deg = degp_ref[0, :, 0:1] + degp_ref[1, :, 0:1] + 1.0
    dinv = lax.rsqrt(deg)
    u = dinv * x_ref[...]
    o_ref[...] = jnp.concatenate(
        [u, dinv, jnp.zeros((_TBLK, Q - 3), jnp.float32)], axis=1)
  return pl.pallas_call(
      bodyfn,
      grid=(N // _TBLK,),
      in_specs=[
          pl.BlockSpec((NC, _TBLK, Q), lambda i: (0, i, 0)),
          pl.BlockSpec((_TBLK, IN_DIM), lambda i: (i, 0)),
      ],
      out_specs=pl.BlockSpec((_TBLK, Q), lambda i: (i, 0)),
      out_shape=jax.ShapeDtypeStruct((N, Q), jnp.float32),
  )(degp, x)


def _tc_h1v(aggp, u16, W1, b1):
  def bodyfn(aggp_ref, u16_ref, w1_ref, b1_ref, o0, o1, o2, o3):
    su = (aggp_ref[0, :, 0:IN_DIM] + aggp_ref[1, :, 0:IN_DIM]
          + u16_ref[:, 0:IN_DIM])
    xw = jnp.dot(su, w1_ref[...], preferred_element_type=jnp.float32)
    dinv = u16_ref[:, 2:3]
    h1 = jnp.maximum(dinv * xw + b1_ref[...], 0.0)
    v = dinv * h1
    o0[...] = v[:, 0:16]
    o1[...] = v[:, 16:32]
    o2[...] = v[:, 32:48]
    o3[...] = v[:, 48:64]
  qspec = pl.BlockSpec((_TBLK, Q), lambda i: (i, 0))
  return pl.pallas_call(
      bodyfn,
      grid=(N // _TBLK,),
      in_specs=[
          pl.BlockSpec((NC, _TBLK, Q), lambda i: (0, i, 0)),
          pl.BlockSpec((_TBLK, Q), lambda i: (i, 0)),
          pl.BlockSpec((IN_DIM, H), lambda i: (0, 0)),
          pl.BlockSpec((1, H), lambda i: (0, 0)),
      ],
      out_specs=[qspec, qspec, qspec, qspec],
      out_shape=[jax.ShapeDtypeStruct((N, Q), jnp.float32)] * 4,
  )(aggp, u16, W1, b1)


def _tc_h2(aq, v0, v1, v2, v3, u16, W2, b2):
  blk = 2048
  nin = (N + blk - 1) // blk - 1
  def bodyfn(aq_ref, v0r, v1r, v2r, v3r, u16_ref, w2_ref, b2_ref, o_ref):
    g = jnp.concatenate(
        [aq_ref[0] + v0r[...], aq_ref[1] + v1r[...],
         aq_ref[2] + v2r[...], aq_ref[3] + v3r[...]], axis=1)
    dinv = u16_ref[:, 2:3]
    hw = jnp.dot(g, w2_ref[...], preferred_element_type=jnp.float32)
    o_ref[...] = jnp.maximum(dinv * hw + b2_ref[...], 0.0)
  qspec = pl.BlockSpec((blk, Q), lambda i: (jnp.minimum(i, nin), 0))
  return pl.pallas_call(
      bodyfn,
      grid=(NP // blk,),
      in_specs=[
          pl.BlockSpec((NQ, blk, Q), lambda i: (0, jnp.minimum(i, nin), 0)),
          qspec, qspec, qspec, qspec,
          qspec,
          pl.BlockSpec((H, H), lambda i: (0, 0)),
          pl.BlockSpec((1, H), lambda i: (0, 0)),
      ],
      out_specs=pl.BlockSpec((blk, H), lambda i: (i, 0)),
      out_shape=jax.ShapeDtypeStruct((NP, H), jnp.float32),
  )(aq, v0, v1, v2, v3, u16, W2, b2)


def _tc_head(sums_p, maxs_p, cnts_p, cwt, cb, f1w, f1b, f2w, f2b):
  def bodyfn(sums_ref, maxs_ref, cnts_ref, cwt_ref, cb_ref,
             f1w_ref, f1b_ref, f2w_ref, f2b_ref, o_ref):
    sums = jnp.sum(sums_ref[...], axis=0)
    maxs = jnp.max(maxs_ref[...], axis=0)
    cnts = jnp.sum(cnts_ref[:, :, 0], axis=0)
    means = sums / jnp.maximum(cnts, 1.0)[:, None]
    feats = jnp.concatenate([means, maxs], axis=1)
    logits = []
    for b in range(NB):
      M = feats[b * LVL:(b + 1) * LVL, :]
      Su = jnp.concatenate(
          [jnp.zeros((1, 2 * H), jnp.float32), M[:-1]], axis=0)
      Sd = jnp.concatenate(
          [M[1:], jnp.zeros((1, 2 * H), jnp.float32)], axis=0)
      conv = (jnp.dot(Su, cwt_ref[0], preferred_element_type=jnp.float32)
              + jnp.dot(M, cwt_ref[1], preferred_element_type=jnp.float32)
              + jnp.dot(Sd, cwt_ref[2], preferred_element_type=jnp.float32))
      conv = jnp.maximum(conv + cb_ref[...], 0.0)
      pooled = jnp.mean(conv, axis=0, keepdims=True)
      z = jnp.maximum(
          jnp.dot(pooled, f1w_ref[...], preferred_element_type=jnp.float32)
          + f1b_ref[...], 0.0)
      logits.append(
          jnp.dot(z, f2w_ref[...], preferred_element_type=jnp.float32)
          + f2b_ref[...])
    o_ref[...] = jax.nn.sigmoid(jnp.concatenate(logits, axis=1))
  return pl.pallas_call(
      bodyfn,
      out_shape=jax.ShapeDtypeStruct((1, NB), jnp.float32),
  )(sums_p, maxs_p, cnts_p, cwt, cb, f1w, f1b, f2w, f2b)


def kernel(x, edge_index, node_depth, batch_index, W1, b1, W2, b2,
           conv_w, conv_b, fc1_w, fc1_b, fc2_w, fc2_b):
  src_rs = edge_index[0].reshape(E // WSUB, WSUB)
  dst_rs = edge_index[1].reshape(E // WSUB, WSUB)

  degp = _sc_deg(dst_rs)
  u16 = _tc_u16(degp, x)
  aggp = _sc_agg16(src_rs, dst_rs, u16)
  v0, v1, v2, v3 = _tc_h1v(aggp, u16, W1, b1.reshape(1, H))
  aq = _sc_agg64(src_rs, dst_rs, v0, v1, v2, v3)
  h2p = _tc_h2(aq, v0, v1, v2, v3, u16, W2, b2.reshape(1, H))

  dep_p = jnp.zeros((NP,), jnp.int32).at[:N].set(node_depth)
  bat_p = jnp.full((NP,), NB, jnp.int32).at[:N].set(batch_index)

  sums_p, maxs_p, cnts_p = _sc_pool(h2p, dep_p, bat_p)

  out = _tc_head(sums_p, maxs_p, cnts_p,
                 conv_w.transpose(2, 1, 0), conv_b.reshape(1, CC),
                 fc1_w, fc1_b.reshape(1, CC), fc2_w, fc2_b.reshape(1, 1))
  return out.reshape(-1)


# speedup vs baseline: 28.0591x; 1.1199x over previous
import jax
import jax.numpy as jnp
from jax import lax
from jax.experimental import pallas as pl
from jax.experimental.pallas import tpu as pltpu
from jax.experimental.pallas import tpu_sc as plsc

N = 100000
E = 1600000
IN_DIM = 2
H = 64
CC = 64
LVL = 128
NB = 4
SEG = NB * LVL
SEGP = 528

NC = 2
NS = 16
NW = NC * NS

Q = 16
NQ = 4

WSUB = 125
RSUB = 5
CHUNK = RSUB * WSUB

ROWS_PER_TILE = N // NS
ZROWS = 625

NP = 102400
NODES_PER_TILE = NP // NW
PCHUNK = 640

_mesh = plsc.VectorSubcoreMesh(core_axis_name="c", subcore_axis_name="s")


def _zero_fill(buf, nrows):
  def zb(i, carry):
    buf[i, :] = jnp.zeros((Q,), jnp.float32)
    return carry
  lax.fori_loop(0, nrows, zb, None)


def _zero_acc(acc, zbuf, s):
  for p in range(ROWS_PER_TILE // ZROWS):
    pltpu.sync_copy(zbuf, acc.at[pl.ds(s * ROWS_PER_TILE + p * ZROWS, ZROWS)])


def _dump_acc(acc, outp, q, s):
  for p in range(ROWS_PER_TILE // ZROWS):
    off = s * ROWS_PER_TILE + p * ZROWS
    pltpu.sync_copy(acc.at[pl.ds(off, ZROWS)], outp.at[q, pl.ds(off, ZROWS)])


def _edge_pipeline(nch, row0, src_rs, dst_rs, tab, acc, idxr, rows,
                   sem_st, sem_g, sem_sc, drain_hbm):
  def stage(ch, slot):
    r0 = row0 + ch * RSUB
    pltpu.async_copy(src_rs.at[pl.ds(r0, RSUB)], idxr.at[slot, 0], sem_st)
    pltpu.async_copy(dst_rs.at[pl.ds(r0, RSUB)], idxr.at[slot, 1], sem_st)

  stage(0, 0)

  def body(ch, carry):
    slot = lax.rem(ch, 3)
    p = lax.rem(ch, 2)
    @pl.when(ch >= 2)
    def _():
      pltpu.make_async_copy(drain_hbm, rows.at[0], sem_sc).wait()
    pltpu.make_async_copy(src_rs.at[pl.ds(0, RSUB)], idxr.at[0, 0], sem_st).wait()
    pltpu.make_async_copy(src_rs.at[pl.ds(0, RSUB)], idxr.at[0, 0], sem_st).wait()
    for r in range(RSUB):
      pltpu.async_copy(tab.at[idxr.at[slot, 0, r]],
                       rows.at[p, pl.ds(r * WSUB, WSUB)], sem_g)
    @pl.when(ch + 1 < nch)
    def _():
      stage(ch + 1, lax.rem(ch + 1, 3))
    pltpu.make_async_copy(drain_hbm, rows.at[0], sem_g).wait()
    for r in range(RSUB):
      pltpu.async_copy(rows.at[p, pl.ds(r * WSUB, WSUB)],
                       acc.at[idxr.at[slot, 1, r]], sem_sc, add=True)
    return carry

  lax.fori_loop(0, nch, body, None)
  pltpu.make_async_copy(drain_hbm, rows.at[0], sem_sc).wait()
  pltpu.make_async_copy(drain_hbm, rows.at[0], sem_sc).wait()


def _sc_deg(dst_rs):
  ept = E // NC // NS
  nch = ept // CHUNK

  def body(dst_rs, outp, acc, didxr, ones_rows, zbuf, sem_st, sem_sc):
    c = lax.axis_index("c")
    s = lax.axis_index("s")
    _zero_fill(zbuf, ZROWS)
    def ob(i, carry):
      ones_rows[i, :] = (1 - jnp.minimum(lax.iota(jnp.int32, Q), 1)).astype(jnp.float32)
      return carry
    lax.fori_loop(0, WSUB, ob, None)
    _zero_acc(acc, zbuf, s)
    plsc.subcore_barrier()
    row0 = (c * (E // NC) + s * ept) // WSUB
    drain_hbm = outp.at[0, pl.ds(0, CHUNK)]

    def stage(ch, slot):
      pltpu.async_copy(dst_rs.at[pl.ds(row0 + ch * RSUB, RSUB)],
                       didxr.at[slot], sem_st)
    stage(0, 0)

    def chunk(ch, carry):
      slot = lax.rem(ch, 3)
      @pl.when(ch >= 2)
      def _():
        pltpu.make_async_copy(drain_hbm, zbuf, sem_sc).wait()
      pltpu.make_async_copy(dst_rs.at[pl.ds(0, RSUB)], didxr.at[0], sem_st).wait()
      @pl.when(ch + 1 < nch)
      def _():
        stage(ch + 1, lax.rem(ch + 1, 3))
      for r in range(RSUB):
        pltpu.async_copy(ones_rows, acc.at[didxr.at[slot, r]], sem_sc, add=True)
      return carry
    lax.fori_loop(0, nch, chunk, None)
    pltpu.make_async_copy(drain_hbm, zbuf, sem_sc).wait()
    pltpu.make_async_copy(drain_hbm, zbuf, sem_sc).wait()
    plsc.subcore_barrier()
    _dump_acc(acc, outp, c, s)

  return pl.kernel(
      body,
      out_type=jax.ShapeDtypeStruct((NC, N, Q), jnp.float32),
      mesh=_mesh,
      compiler_params=pltpu.CompilerParams(use_tc_tiling_on_sc=False),
      scratch_types=[
          pltpu.VMEM_SHARED((N, Q), jnp.float32),
          pltpu.VMEM((3, RSUB, WSUB), jnp.int32),
          pltpu.VMEM((WSUB, Q), jnp.float32),
          pltpu.VMEM((ZROWS, Q), jnp.float32),
          pltpu.SemaphoreType.DMA,
          pltpu.SemaphoreType.DMA,
      ],
  )(dst_rs)


def _sc_agg16(src_rs, dst_rs, tab):
  ept = E // NC // NS
  nch = ept // CHUNK

  def body(src_rs, dst_rs, tab, outp, acc, idxr, rows,
           sem_st, sem_g, sem_sc):
    c = lax.axis_index("c")
    s = lax.axis_index("s")
    _zero_fill(rows.at[0], CHUNK)
    _zero_acc(acc, rows.at[0, pl.ds(0, ZROWS)], s)
    plsc.subcore_barrier()
    row0 = (c * (E // NC) + s * ept) // WSUB
    _edge_pipeline(nch, row0, src_rs, dst_rs, tab, acc, idxr, rows,
                   sem_st, sem_g, sem_sc, tab.at[pl.ds(0, CHUNK)])
    plsc.subcore_barrier()
    _dump_acc(acc, outp, c, s)

  return pl.kernel(
      body,
      out_type=jax.ShapeDtypeStruct((NC, N, Q), jnp.float32),
      mesh=_mesh,
      compiler_params=pltpu.CompilerParams(use_tc_tiling_on_sc=False),
      scratch_types=[
          pltpu.VMEM_SHARED((N, Q), jnp.float32),
          pltpu.VMEM((3, 2, RSUB, WSUB), jnp.int32),
          pltpu.VMEM((2, CHUNK, Q), jnp.float32),
          pltpu.SemaphoreType.DMA,
          pltpu.SemaphoreType.DMA,
          pltpu.SemaphoreType.DMA,
      ],
  )(src_rs, dst_rs, tab)


def _sc_agg64(src_rs, dst_rs, v0, v1, v2, v3):
  ept = E // NS
  nch = ept // CHUNK

  def body(src_rs, dst_rs, v0, v1, v2, v3, outp,
           acc, idxr, rows, sem_st, sem_g, sem_sc):
    c = lax.axis_index("c")
    s = lax.axis_index("s")
    row0 = s * ept // WSUB

    for rnd in range(2):
      _zero_fill(rows.at[0], CHUNK)
      _zero_acc(acc, rows.at[0, pl.ds(0, ZROWS)], s)
      plsc.subcore_barrier()
      ta = v0 if rnd == 0 else v2
      tb = v1 if rnd == 0 else v3
      @pl.when(c == 0)
      def _():
        _edge_pipeline(nch, row0, src_rs, dst_rs, ta, acc, idxr, rows,
                       sem_st, sem_g, sem_sc, ta.at[pl.ds(0, CHUNK)])
      @pl.when(c == 1)
      def _():
        _edge_pipeline(nch, row0, src_rs, dst_rs, tb, acc, idxr, rows,
                       sem_st, sem_g, sem_sc, tb.at[pl.ds(0, CHUNK)])
      plsc.subcore_barrier()
      _dump_acc(acc, outp, 2 * rnd + c, s)
      plsc.subcore_barrier()

  return pl.kernel(
      body,
      out_type=jax.ShapeDtypeStruct((NQ, N, Q), jnp.float32),
      mesh=_mesh,
      compiler_params=pltpu.CompilerParams(use_tc_tiling_on_sc=False),
      scratch_types=[
          pltpu.VMEM_SHARED((N, Q), jnp.float32),
          pltpu.VMEM((3, 2, RSUB, WSUB), jnp.int32),
          pltpu.VMEM((2, CHUNK, Q), jnp.float32),
          pltpu.SemaphoreType.DMA,
          pltpu.SemaphoreType.DMA,
          pltpu.SemaphoreType.DMA,
      ],
  )(src_rs, dst_rs, v0, v1, v2, v3)


def _sc_pool(h2p, dep_p, bat_p):
  nch = NODES_PER_TILE // PCHUNK

  def body(h2p, dep_p, bat_p, sums_o, maxs_o, cnts_o,
           sums, maxs, cnts, hbuf, dbuf, bbuf):
    c = lax.axis_index("c")
    s = lax.axis_index("s")
    wid = s * NC + c
    e0 = (1 - jnp.minimum(lax.iota(jnp.int32, Q), 1)).astype(jnp.float32)
    def za(i, carry):
      for j in range(4):
        sums[i, pl.ds(16 * j, 16)] = jnp.zeros((16,), jnp.float32)
        maxs[i, pl.ds(16 * j, 16)] = jnp.zeros((16,), jnp.float32)
      cnts[i, :] = jnp.zeros((16,), jnp.float32)
      return carry
    lax.fori_loop(0, SEGP, za, None)

    def chunk(ch, carry):
      base = wid * NODES_PER_TILE + ch * PCHUNK
      pltpu.sync_copy(h2p.at[pl.ds(base, PCHUNK)], hbuf)
      pltpu.sync_copy(dep_p.at[pl.ds(base, PCHUNK)], dbuf.at[pl.ds(0, PCHUNK)])
      pltpu.sync_copy(bat_p.at[pl.ds(base, PCHUNK)], bbuf.at[pl.ds(0, PCHUNK)])
      def node(i, carry2):
        d = dbuf[pl.ds(i, 16)][0]
        b = bbuf[pl.ds(i, 16)][0]
        sgi = b * LVL + jnp.clip(d, 0, LVL - 1)
        for j in range(4):
          hv = hbuf[i, pl.ds(16 * j, 16)]
          sums[sgi, pl.ds(16 * j, 16)] = sums[sgi, pl.ds(16 * j, 16)] + hv
          maxs[sgi, pl.ds(16 * j, 16)] = jnp.maximum(
              maxs[sgi, pl.ds(16 * j, 16)], hv)
        cnts[sgi, :] = cnts[sgi, :] + e0
        return carry2
      lax.fori_loop(0, PCHUNK, node, None)
      return carry
    lax.fori_loop(0, nch, chunk, None)

    pltpu.sync_copy(sums.at[pl.ds(0, SEG)], sums_o.at[wid])
    pltpu.sync_copy(maxs.at[pl.ds(0, SEG)], maxs_o.at[wid])
    pltpu.sync_copy(cnts.at[pl.ds(0, SEG)], cnts_o.at[wid])

  return pl.kernel(
      body,
      out_type=[
          jax.ShapeDtypeStruct((NW, SEG, H), jnp.float32),
          jax.ShapeDtypeStruct((NW, SEG, H), jnp.float32),
          jax.ShapeDtypeStruct((NW, SEG, Q), jnp.float32),
      ],
      mesh=_mesh,
      compiler_params=pltpu.CompilerParams(use_tc_tiling_on_sc=False),
      scratch_types=[
          pltpu.VMEM((SEGP, H), jnp.float32),
          pltpu.VMEM((SEGP, H), jnp.float32),
          pltpu.VMEM((SEGP, Q), jnp.float32),
          pltpu.VMEM((PCHUNK, H), jnp.float32),
          pltpu.VMEM((PCHUNK + 16,), jnp.int32),
          pltpu.VMEM((PCHUNK + 16,), jnp.int32),
      ],
  )(h2p, dep_p, bat_p)


_TBLK = 2000


def _tc_u16(degp, x):
  def bodyfn(degp_ref, x_ref, o_ref):
    deg = degp_ref[0, :, 0:1] + degp_ref[1, :, 0:1] + 1.0
    dinv = lax.rsqrt(deg)
    u = dinv * x_ref[...]
    o_ref[...] = jnp.concatenate(
        [u, dinv, jnp.zeros((_TBLK, Q - 3), jnp.float32)], axis=1)
  return pl.pallas_call(
      bodyfn,
      grid=(N // _TBLK,),
      in_specs=[
          pl.BlockSpec((NC, _TBLK, Q), lambda i: (0, i, 0)),
          pl.BlockSpec((_TBLK, IN_DIM), lambda i: (i, 0)),
      ],
      out_specs=pl.BlockSpec((_TBLK, Q), lambda i: (i, 0)),
      out_shape=jax.ShapeDtypeStruct((N, Q), jnp.float32),
  )(degp, x)


def _tc_h1v(aggp, u16, W1, b1):
  def bodyfn(aggp_ref, u16_ref, w1_ref, b1_ref, o0, o1, o2, o3):
    su = (aggp_ref[0, :, 0:IN_DIM] + aggp_ref[1, :, 0:IN_DIM]
          + u16_ref[:, 0:IN_DIM])
    xw = jnp.dot(su, w1_ref[...], preferred_element_type=jnp.float32)
    dinv = u16_ref[:, 2:3]
    h1 = jnp.maximum(dinv * xw + b1_ref[...], 0.0)
    v = dinv * h1
    o0[...] = v[:, 0:16]
    o1[...] = v[:, 16:32]
    o2[...] = v[:, 32:48]
    o3[...] = v[:, 48:64]
  qspec = pl.BlockSpec((_TBLK, Q), lambda i: (i, 0))
  return pl.pallas_call(
      bodyfn,
      grid=(N // _TBLK,),
      in_specs=[
          pl.BlockSpec((NC, _TBLK, Q), lambda i: (0, i, 0)),
          pl.BlockSpec((_TBLK, Q), lambda i: (i, 0)),
          pl.BlockSpec((IN_DIM, H), lambda i: (0, 0)),
          pl.BlockSpec((1, H), lambda i: (0, 0)),
      ],
      out_specs=[qspec, qspec, qspec, qspec],
      out_shape=[jax.ShapeDtypeStruct((N, Q), jnp.float32)] * 4,
  )(aggp, u16, W1, b1)


def _tc_h2(aqP, v0P, v1P, v2P, v3P, u16P, Wbig, S8, B8, b2P):
  blk8 = 256
  nin = (N // 8 + blk8 - 1) // blk8 - 1
  def bodyfn(aq_ref, v0r, v1r, v2r, v3r, u16r,
             wbig_ref, s8_ref, b8_ref, b2_ref, o_ref):
    hw = (jnp.dot(aq_ref[0] + v0r[...], wbig_ref[0],
                  preferred_element_type=jnp.float32)
          + jnp.dot(aq_ref[1] + v1r[...], wbig_ref[1],
                    preferred_element_type=jnp.float32)
          + jnp.dot(aq_ref[2] + v2r[...], wbig_ref[2],
                    preferred_element_type=jnp.float32)
          + jnp.dot(aq_ref[3] + v3r[...], wbig_ref[3],
                    preferred_element_type=jnp.float32))
    dinv8 = jnp.dot(u16r[...], s8_ref[...],
                    preferred_element_type=jnp.float32)
    dinvP = jnp.dot(dinv8, b8_ref[...],
                    preferred_element_type=jnp.float32)
    o_ref[...] = jnp.maximum(dinvP * hw + b2_ref[...], 0.0)
  qspec = pl.BlockSpec((blk8, 128), lambda i: (jnp.minimum(i, nin), 0))
  return pl.pallas_call(
      bodyfn,
      grid=(NP // 8 // blk8,),
      in_specs=[
          pl.BlockSpec((NQ, blk8, 128), lambda i: (0, jnp.minimum(i, nin), 0)),
          qspec, qspec, qspec, qspec,
          qspec,
          pl.BlockSpec((NQ, 128, 8 * H), lambda i: (0, 0, 0)),
          pl.BlockSpec((128, 8), lambda i: (0, 0)),
          pl.BlockSpec((8, 8 * H), lambda i: (0, 0)),
          pl.BlockSpec((1, 8 * H), lambda i: (0, 0)),
      ],
      out_specs=pl.BlockSpec((blk8, 8 * H), lambda i: (i, 0)),
      out_shape=jax.ShapeDtypeStruct((NP // 8, 8 * H), jnp.float32),
  )(aqP, v0P, v1P, v2P, v3P, u16P, Wbig, S8, B8, b2P)


def _tc_head(sums_p, maxs_p, cnts_p, cwt, cb, f1w, f1b, f2w, f2b):
  def bodyfn(sums_ref, maxs_ref, cnts_ref, cwt_ref, cb_ref,
             f1w_ref, f1b_ref, f2w_ref, f2b_ref, o_ref):
    sums = jnp.sum(sums_ref[...], axis=0)
    maxs = jnp.max(maxs_ref[...], axis=0)
    cnts = jnp.sum(cnts_ref[:, :, 0], axis=0)
    means = sums / jnp.maximum(cnts, 1.0)[:, None]
    feats = jnp.concatenate([means, maxs], axis=1)
    logits = []
    for b in range(NB):
      M = feats[b * LVL:(b + 1) * LVL, :]
      Su = jnp.concatenate(
          [jnp.zeros((1, 2 * H), jnp.float32), M[:-1]], axis=0)
      Sd = jnp.concatenate(
          [M[1:], jnp.zeros((1, 2 * H), jnp.float32)], axis=0)
      conv = (jnp.dot(Su, cwt_ref[0], preferred_element_type=jnp.float32)
              + jnp.dot(M, cwt_ref[1], preferred_element_type=jnp.float32)
              + jnp.dot(Sd, cwt_ref[2], preferred_element_type=jnp.float32))
      conv = jnp.maximum(conv + cb_ref[...], 0.0)
      pooled = jnp.mean(conv, axis=0, keepdims=True)
      z = jnp.maximum(
          jnp.dot(pooled, f1w_ref[...], preferred_element_type=jnp.float32)
          + f1b_ref[...], 0.0)
      logits.append(
          jnp.dot(z, f2w_ref[...], preferred_element_type=jnp.float32)
          + f2b_ref[...])
    o_ref[...] = jax.nn.sigmoid(jnp.concatenate(logits, axis=1))
  return pl.pallas_call(
      bodyfn,
      out_shape=jax.ShapeDtypeStruct((1, NB), jnp.float32),
  )(sums_p, maxs_p, cnts_p, cwt, cb, f1w, f1b, f2w, f2b)


def kernel(x, edge_index, node_depth, batch_index, W1, b1, W2, b2,
           conv_w, conv_b, fc1_w, fc1_b, fc2_w, fc2_b):
  src_rs = edge_index[0].reshape(E // WSUB, WSUB)
  dst_rs = edge_index[1].reshape(E // WSUB, WSUB)

  degp = _sc_deg(dst_rs)
  u16 = _tc_u16(degp, x)
  aggp = _sc_agg16(src_rs, dst_rs, u16)
  v0, v1, v2, v3 = _tc_h1v(aggp, u16, W1, b1.reshape(1, H))
  aq = _sc_agg64(src_rs, dst_rs, v0, v1, v2, v3)

  eye8 = jnp.eye(8, dtype=jnp.float32)
  W2r = W2.reshape(NQ, Q, H)
  Wbig = jnp.einsum('qjo,kK->qkjKo', W2r, eye8).reshape(NQ, 128, 8 * H)
  e2 = (jnp.arange(Q) == 2).astype(jnp.float32)
  S8 = jnp.einsum('j,kK->kjK', e2, eye8).reshape(128, 8)
  B8 = jnp.einsum('kK,o->kKo', eye8,
                  jnp.ones((H,), jnp.float32)).reshape(8, 8 * H)
  b2P = jnp.tile(b2, (8,)).reshape(1, 8 * H)

  h2pk = _tc_h2(aq.reshape(NQ, N // 8, 128),
                v0.reshape(N // 8, 128), v1.reshape(N // 8, 128),
                v2.reshape(N // 8, 128), v3.reshape(N // 8, 128),
                u16.reshape(N // 8, 128), Wbig, S8, B8, b2P)
  h2p = h2pk.reshape(NP, H)

  dep_p = jnp.zeros((NP,), jnp.int32).at[:N].set(node_depth)
  bat_p = jnp.full((NP,), NB, jnp.int32).at[:N].set(batch_index)

  sums_p, maxs_p, cnts_p = _sc_pool(h2p, dep_p, bat_p)

  out = _tc_head(sums_p, maxs_p, cnts_p,
                 conv_w.transpose(2, 1, 0), conv_b.reshape(1, CC),
                 fc1_w, fc1_b.reshape(1, CC), fc2_w, fc2_b.reshape(1, 1))
  return out.reshape(-1)
